# Initial kernel scaffold; baseline (speedup 1.0000x reference)
import jax
import jax.numpy as jnp
from jax.experimental import pallas as pl


def kernel(x, edge_index, batch, emb_table, W1, b1, W2, b2, Wl1, bl1, Wl2, bl2):
    raise NotImplementedError("write your pallas kernel here")


# speedup vs baseline: 17.6244x; 17.6244x over previous
import functools

import jax
import jax.numpy as jnp
from jax import lax
from jax.experimental import pallas as pl
from jax.experimental.pallas import tpu as pltpu
from jax.experimental.pallas import tpu_sc as plsc

N = 10000
E = 320000
VOCAB = 10000
D = 128
B = 16
LD = 64

NC = 2
NS = 16
NW = NC * NS

EPW = E // NW
ECH = 80
ENC = EPW // ECH

RCH = 40
RNC = N // RCH
RK = -(-RNC // NW)

RPT = 632
RPT_LAST = N - (NS - 1) * RPT

_mesh = plsc.VectorSubcoreMesh(core_axis_name="c", subcore_axis_name="s")


def _sc_gather_deg_body(t1_hbm, x_hbm, dstr_hbm, z128_hbm, ones_hbm,
                        hw1_hbm, hist_hbm,
                        hist_acc, xin_v, rows_v, din_v, ones_v):
  c = lax.axis_index("c")
  s = lax.axis_index("s")
  wid = c * NS + s
  r0 = s * RPT
  @pl.when(s < NS - 1)
  def _():
    pltpu.sync_copy(z128_hbm, hist_acc.at[pl.ds(r0, RPT)])
  @pl.when(s == NS - 1)
  def _():
    pltpu.sync_copy(z128_hbm.at[pl.ds(0, RPT_LAST)],
                    hist_acc.at[pl.ds(r0, RPT_LAST)])
  pltpu.sync_copy(ones_hbm, ones_v)
  pltpu.sync_copy(dstr_hbm.at[wid], din_v)
  plsc.subcore_barrier()
  @pl.loop(0, RK)
  def _(k):
    cid = wid + k * NW
    @pl.when(cid < RNC)
    def _():
      pltpu.sync_copy(x_hbm.at[pl.ds(cid * RCH, RCH)], xin_v)
      pltpu.sync_copy(t1_hbm.at[xin_v], rows_v)
      pltpu.sync_copy(rows_v, hw1_hbm.at[pl.ds(cid * RCH, RCH)])
  @pl.loop(0, ENC)
  def _(i):
    pltpu.sync_copy(ones_v, hist_acc.at[din_v.at[i]], add=True)
  plsc.subcore_barrier()
  @pl.when(s < NS - 1)
  def _():
    pltpu.sync_copy(hist_acc.at[pl.ds(r0, RPT)],
                    hist_hbm.at[pl.ds(c * N + r0, RPT)])
  @pl.when(s == NS - 1)
  def _():
    pltpu.sync_copy(hist_acc.at[pl.ds(r0, RPT_LAST)],
                    hist_hbm.at[pl.ds(c * N + r0, RPT_LAST)])


_sc_gather_deg = pl.kernel(
    _sc_gather_deg_body,
    out_type=(jax.ShapeDtypeStruct((N, D), jnp.float32),
              jax.ShapeDtypeStruct((NC * N, D), jnp.float32)),
    mesh=_mesh,
    scratch_types=[
        pltpu.VMEM_SHARED((N, D), jnp.float32),
        pltpu.VMEM((RCH,), jnp.int32),
        pltpu.VMEM((RCH, D), jnp.float32),
        pltpu.VMEM((ENC, ECH), jnp.int32),
        pltpu.VMEM((ECH, D), jnp.float32),
    ],
)


def _sc_edge_body(g_hbm, srcr_hbm, dstr_hbm, z128_hbm, out_hbm,
                  acc, sidx_v, didx_v, rows_v):
  c = lax.axis_index("c")
  s = lax.axis_index("s")
  wid = c * NS + s
  r0 = s * RPT
  @pl.when(s < NS - 1)
  def _():
    pltpu.sync_copy(z128_hbm, acc.at[pl.ds(r0, RPT)])
  @pl.when(s == NS - 1)
  def _():
    pltpu.sync_copy(z128_hbm.at[pl.ds(0, RPT_LAST)],
                    acc.at[pl.ds(r0, RPT_LAST)])
  pltpu.sync_copy(srcr_hbm.at[wid], sidx_v)
  pltpu.sync_copy(dstr_hbm.at[wid], didx_v)
  plsc.subcore_barrier()
  @pl.loop(0, ENC)
  def _(i):
    pltpu.sync_copy(g_hbm.at[sidx_v.at[i]], rows_v)
    pltpu.sync_copy(rows_v, acc.at[didx_v.at[i]], add=True)
  plsc.subcore_barrier()
  @pl.when(s < NS - 1)
  def _():
    pltpu.sync_copy(acc.at[pl.ds(r0, RPT)],
                    out_hbm.at[pl.ds(c * N + r0, RPT)])
  @pl.when(s == NS - 1)
  def _():
    pltpu.sync_copy(acc.at[pl.ds(r0, RPT_LAST)],
                    out_hbm.at[pl.ds(c * N + r0, RPT_LAST)])


_sc_edge = pl.kernel(
    _sc_edge_body,
    out_type=jax.ShapeDtypeStruct((NC * N, D), jnp.float32),
    mesh=_mesh,
    scratch_types=[
        pltpu.VMEM_SHARED((N, D), jnp.float32),
        pltpu.VMEM((ENC, ECH), jnp.int32),
        pltpu.VMEM((ENC, ECH), jnp.int32),
        pltpu.VMEM((ECH, D), jnp.float32),
    ],
)


def _tc_t1_body(emb_ref, w1_ref, o_ref):
  o_ref[...] = jnp.dot(emb_ref[...], w1_ref[...],
                       preferred_element_type=jnp.float32)


_tc_t1 = pl.pallas_call(
    _tc_t1_body,
    out_shape=jax.ShapeDtypeStruct((VOCAB, D), jnp.float32),
)


def _tc_scale_body(hw1_ref, hist_ref, g1_ref, dinv_ref):
  deg = 1.0 + hist_ref[0:N, 0:1] + hist_ref[N:2 * N, 0:1]
  dinv = lax.rsqrt(deg)
  dinv_ref[...] = dinv
  g1_ref[...] = hw1_ref[...] * dinv


_tc_scale = pl.pallas_call(
    _tc_scale_body,
    out_shape=(jax.ShapeDtypeStruct((N, D), jnp.float32),
               jax.ShapeDtypeStruct((N, 1), jnp.float32)),
)


def _tc_layer2_body(s1_ref, g1_ref, dinv_ref, b1_ref, w2_ref, g2_ref):
  dinv = dinv_ref[...]
  h1 = jnp.maximum(
      dinv * (s1_ref[0:N] + s1_ref[N:2 * N] + g1_ref[...]) + b1_ref[...], 0.0)
  hw2 = jnp.dot(h1, w2_ref[...], preferred_element_type=jnp.float32)
  g2_ref[...] = dinv * hw2


_tc_layer2 = pl.pallas_call(
    _tc_layer2_body,
    out_shape=jax.ShapeDtypeStruct((N, D), jnp.float32),
)


def _tc_final_body(s2_ref, g2_ref, dinv_ref, b2_ref, batch_ref,
                   wl1_ref, bl1_ref, wl2_ref, bl2_ref, o_ref):
  dinv = dinv_ref[...]
  h2 = dinv * (s2_ref[0:N] + s2_ref[N:2 * N] + g2_ref[...]) + b2_ref[...]
  iot = lax.broadcasted_iota(jnp.int32, (B, N), 0)
  bm = (jnp.broadcast_to(batch_ref[...], (B, N)) == iot).astype(jnp.float32)
  ssum = jnp.dot(bm, h2, preferred_element_type=jnp.float32)
  cnt = jnp.sum(bm, axis=1, keepdims=True)
  pooled = ssum / jnp.maximum(cnt, 1.0)
  z = jnp.maximum(
      jnp.dot(pooled, wl1_ref[...], preferred_element_type=jnp.float32)
      + bl1_ref[...], 0.0)
  t = (jnp.dot(z, wl2_ref[...], preferred_element_type=jnp.float32)
       + bl2_ref[...])
  o_ref[...] = 1.0 / (1.0 + jnp.exp(-t))


_tc_final = pl.pallas_call(
    _tc_final_body,
    out_shape=jax.ShapeDtypeStruct((B, 1), jnp.float32),
)


def kernel(x, edge_index, batch, emb_table, W1, b1, W2, b2, Wl1, bl1, Wl2, bl2):
  x = x.astype(jnp.int32)
  src = edge_index[0].astype(jnp.int32).reshape(NW, ENC, ECH)
  dst = edge_index[1].astype(jnp.int32).reshape(NW, ENC, ECH)
  z128 = jnp.zeros((RPT, D), jnp.float32)
  ones16 = jnp.ones((ECH, D), jnp.float32)

  t1 = _tc_t1(emb_table, W1)
  hw1, hist = _sc_gather_deg(t1, x, dst, z128, ones16)
  g1, dinv = _tc_scale(hw1, hist)
  s1 = _sc_edge(g1, src, dst, z128)
  g2 = _tc_layer2(s1, g1, dinv, b1.reshape(1, D), W2)
  s2 = _sc_edge(g2, src, dst, z128)
  out = _tc_final(s2, g2, dinv, b2.reshape(1, D),
                  batch.astype(jnp.int32).reshape(1, N),
                  Wl1, bl1.reshape(1, LD), Wl2, bl2.reshape(1, 1))
  return out
